# baseline (speedup 1.0000x reference)
import jax
import jax.numpy as jnp
from jax.experimental import pallas as pl
from jax.experimental.pallas import tpu as pltpu

_TOKENS = 4 * 8192
_CH = 1024
_BLK_T = 512
_GRID = _TOKENS // _BLK_T


def _masked_mse_kernel(p_ref, t_ref, m_ref, loss_ref, acc_ref, cnt_ref):
    i = pl.program_id(0)

    @pl.when(i == 0)
    def _init():
        acc_ref[...] = jnp.zeros((8, 128), jnp.float32)
        cnt_ref[0] = 0.0

    m = m_ref[0]
    d = (p_ref[...] - t_ref[...]) * m
    sq = d * d
    acc_ref[...] += jnp.sum(sq.reshape(-1, 8, 128), axis=0)
    cnt_ref[0] += jnp.sum(m)

    @pl.when(i == _GRID - 1)
    def _fin():
        n = cnt_ref[0] * _CH
        loss_ref[0, 0] = jnp.sum(acc_ref[...]) / jnp.maximum(n, 1.0)


def kernel(predicted, target, mask):
    tgt_dim = target.shape[-1]
    pred = predicted[..., :tgt_dim].reshape(_TOKENS, _CH)
    tgt = target.reshape(_TOKENS, _CH)
    m = mask.reshape(_GRID, _BLK_T, 1).astype(jnp.float32)

    loss = pl.pallas_call(
        _masked_mse_kernel,
        grid=(_GRID,),
        in_specs=[
            pl.BlockSpec((_BLK_T, _CH), lambda i: (i, 0)),
            pl.BlockSpec((_BLK_T, _CH), lambda i: (i, 0)),
            pl.BlockSpec((1, _BLK_T, 1), lambda i: (i, 0, 0)),
        ],
        out_specs=pl.BlockSpec(memory_space=pltpu.SMEM),
        out_shape=jax.ShapeDtypeStruct((1, 1), jnp.float32),
        scratch_shapes=[
            pltpu.VMEM((8, 128), jnp.float32),
            pltpu.SMEM((1,), jnp.float32),
        ],
    )(pred, tgt, m)
    return loss[0, 0]


# speedup vs baseline: 1.5772x; 1.5772x over previous
import jax
import jax.numpy as jnp
from jax.experimental import pallas as pl
from jax.experimental.pallas import tpu as pltpu

_TOKENS = 4 * 8192
_CH = 1024
_BLK_T = 1024
_GRID = _TOKENS // _BLK_T


def _masked_mse_kernel(p_ref, t_ref, m_ref, loss_ref, acc_ref, cnt_ref):
    i = pl.program_id(0)

    @pl.when(i == 0)
    def _init():
        acc_ref[0] = 0.0
        cnt_ref[0] = 0.0

    d = p_ref[...] - t_ref[...]
    m = m_ref[0, 0]
    row_sq = jnp.sum(d * d, axis=1)
    acc_ref[0] += jnp.sum(row_sq * m)
    cnt_ref[0] += jnp.sum(m)

    @pl.when(i == _GRID - 1)
    def _fin():
        n = cnt_ref[0] * _CH
        loss_ref[0, 0] = acc_ref[0] / jnp.maximum(n, 1.0)


def kernel(predicted, target, mask):
    tgt_dim = target.shape[-1]
    pred = predicted[..., :tgt_dim].reshape(_TOKENS, _CH)
    tgt = target.reshape(_TOKENS, _CH)
    m = mask.reshape(_GRID, 1, _BLK_T).astype(jnp.float32)

    loss = pl.pallas_call(
        _masked_mse_kernel,
        grid=(_GRID,),
        in_specs=[
            pl.BlockSpec((_BLK_T, _CH), lambda i: (i, 0)),
            pl.BlockSpec((_BLK_T, _CH), lambda i: (i, 0)),
            pl.BlockSpec((1, 1, _BLK_T), lambda i: (i, 0, 0)),
        ],
        out_specs=pl.BlockSpec(memory_space=pltpu.SMEM),
        out_shape=jax.ShapeDtypeStruct((1, 1), jnp.float32),
        scratch_shapes=[
            pltpu.SMEM((1,), jnp.float32),
            pltpu.SMEM((1,), jnp.float32),
        ],
    )(pred, tgt, m)
    return loss[0, 0]
